# baseline (speedup 1.0000x reference)
import functools

import jax
import jax.numpy as jnp
from jax import lax
from jax.experimental import pallas as pl
from jax.experimental.pallas import tpu as pltpu
from jax.experimental.pallas import tpu_sc as plsc

VOCAB = 1000000 + 999 + 1
SEQ = 200
EMB = 64
BATCH = 4096
EPS = 1e-12

CHUNK = 100
CHUNK_PAD = 104
NCHUNK = BATCH * SEQ // CHUNK
NWORKERS = 32
PER_W = NCHUNK // NWORKERS


def _lane_sum(v):
    iota = lax.iota(jnp.int32, 16)
    for k in (1, 2, 4, 8):
        v = v + v.at[iota ^ k].get(mode="promise_in_bounds")
    return v


def _rsqrt_newton(x):
    xi = lax.bitcast_convert_type(x, jnp.int32)
    yi = jnp.int32(0x5F3759DF) - (xi >> 1)
    y = lax.bitcast_convert_type(yi, jnp.float32)
    for _ in range(3):
        y = y * (1.5 - 0.5 * x * y * y)
    return y


def _sc_body(word_hbm, idx1_hbm, idx2_hbm, pos_hbm, gam_hbm, bet_hbm,
             out_hbm,
             idx1_v, idx2_v, rows1_v, rows2_v, out_v, pos_v, gam_v, bet_v,
             sem):
    wid = lax.axis_index("s") * 2 + lax.axis_index("c")
    base = wid * PER_W

    pltpu.sync_copy(pos_hbm, pos_v)
    pltpu.sync_copy(gam_hbm, gam_v)
    pltpu.sync_copy(bet_hbm, bet_v)

    gvec = [gam_v[pl.ds(16 * i, 16)] for i in range(4)]
    bvec = [bet_v[pl.ds(16 * i, 16)] for i in range(4)]
    inv_e = jnp.float32(1.0 / EMB)

    def do_chunk(k, carry):
        c = base + k
        pos_off = (c % 2) * CHUNK

        pltpu.sync_copy(idx1_hbm.at[c], idx1_v)
        pltpu.sync_copy(idx2_hbm.at[c], idx2_v)
        cp1 = pltpu.async_copy(word_hbm.at[idx1_v], rows1_v, sem)
        cp2 = pltpu.async_copy(word_hbm.at[idx2_v], rows2_v, sem)
        cp1.wait()
        cp2.wait()

        def token(t, _):
            e = []
            for i in range(4):
                a = rows1_v[t, pl.ds(16 * i, 16)]
                b = rows2_v[t, pl.ds(16 * i, 16)]
                p = pos_v[pos_off + t, pl.ds(16 * i, 16)]
                e.append(a + b + p)
            tot = _lane_sum((e[0] + e[1]) + (e[2] + e[3]))
            mean = tot * inv_e
            d = [ei - mean for ei in e]
            q = (d[0] * d[0] + d[1] * d[1]) + (d[2] * d[2] + d[3] * d[3])
            var = _lane_sum(q) * inv_e
            scale = _rsqrt_newton(var + EPS)
            for i in range(4):
                out_v[t, pl.ds(16 * i, 16)] = (d[i] * scale) * gvec[i] + bvec[i]
            return _

        lax.fori_loop(0, CHUNK, token, 0, unroll=False)
        pltpu.sync_copy(out_v, out_hbm.at[c])
        return carry

    lax.fori_loop(0, PER_W, do_chunk, 0, unroll=False)


@functools.partial(jax.jit, static_argnames=())
def _run(word, idx1p, idx2p, position, gamma, beta):
    mesh = plsc.VectorSubcoreMesh(core_axis_name="c", subcore_axis_name="s")
    kfn = pl.kernel(
        _sc_body,
        mesh=mesh,
        compiler_params=pltpu.CompilerParams(use_tc_tiling_on_sc=False),
        out_type=jax.ShapeDtypeStruct((NCHUNK, CHUNK, EMB), jnp.float32),
        scratch_types=[
            pltpu.VMEM((CHUNK_PAD,), jnp.int32),
            pltpu.VMEM((CHUNK_PAD,), jnp.int32),
            pltpu.VMEM((CHUNK_PAD, EMB), jnp.float32),
            pltpu.VMEM((CHUNK_PAD, EMB), jnp.float32),
            pltpu.VMEM((CHUNK, EMB), jnp.float32),
            pltpu.VMEM((SEQ, EMB), jnp.float32),
            pltpu.VMEM((EMB,), jnp.float32),
            pltpu.VMEM((EMB,), jnp.float32),
            pltpu.SemaphoreType.DMA,
        ],
    )
    return kfn(word, idx1p, idx2p, position, gamma, beta)


def kernel(_input, word, position, gamma, beta):
    vocab_size = word.shape[0]
    idx = ((_input + vocab_size) % vocab_size).astype(jnp.int32)
    idx1 = idx[:, :, 0].reshape(NCHUNK, CHUNK)
    idx2 = idx[:, :, 1].reshape(NCHUNK, CHUNK)
    pad = ((0, 0), (0, CHUNK_PAD - CHUNK))
    idx1p = jnp.pad(idx1, pad)
    idx2p = jnp.pad(idx2, pad)
    out = _run(word, idx1p, idx2p, position, gamma, beta)
    return out.reshape(BATCH, SEQ, EMB)
